# baseline (speedup 1.0000x reference)
import functools

import jax
import jax.numpy as jnp
from jax import lax
from jax.experimental import pallas as pl
from jax.experimental.pallas import tpu as pltpu
from jax.experimental.pallas import tpu_sc as plsc

_NC = 2
_NS = 16
_NW = _NC * _NS
_L = 16
_TPAD = 160
_CHUNK = 8192


def _make_lookup(n_img, plane, n_imap):
    per_tile = (n_img * plane) // _NW
    tiles_per_img = plane // per_tile
    n_chunks = per_tile // _CHUNK
    assert per_tile % _CHUNK == 0 and plane % per_tile == 0

    mesh = plsc.VectorSubcoreMesh(core_axis_name="c", subcore_axis_name="s")

    @functools.partial(
        pl.kernel,
        out_type=jax.ShapeDtypeStruct((n_img * 3 * plane,), jnp.float32),
        mesh=mesh,
        compiler_params=pltpu.CompilerParams(needs_layout_passes=False),
        scratch_types=[
            pltpu.VMEM((_TPAD,), jnp.int32),
            pltpu.VMEM((216, 3), jnp.float32),
            pltpu.VMEM((_TPAD,), jnp.float32),
            pltpu.VMEM((_TPAD,), jnp.float32),
            pltpu.VMEM((_TPAD,), jnp.float32),
            pltpu.VMEM((_CHUNK,), jnp.int32),
            pltpu.VMEM((_CHUNK,), jnp.int32),
            pltpu.VMEM((_CHUNK,), jnp.float32),
            pltpu.VMEM((_CHUNK,), jnp.float32),
            pltpu.VMEM((_CHUNK,), jnp.float32),
            pltpu.VMEM((_CHUNK,), jnp.float32),
            pltpu.VMEM((_CHUNK,), jnp.float32),
            pltpu.VMEM((_CHUNK,), jnp.float32),
            pltpu.SemaphoreType.DMA,
            pltpu.SemaphoreType.DMA,
            pltpu.SemaphoreType.DMA,
            pltpu.SemaphoreType.DMA,
        ],
    )
    def lookup(mask_hbm, mesh_hbm, imap_hbm, out_hbm,
               imap_v, mesh_v, t0, t1, t2, mca, mcb,
               o0a, o1a, o2a, o0b, o1b, o2b, sia, sib, soa, sob):
        wid = lax.axis_index("s") * _NC + lax.axis_index("c")
        img = wid // tiles_per_img
        qoff = (wid % tiles_per_img) * per_tile
        masks = (mca, mcb)
        outs = ((o0a, o1a, o2a), (o0b, o1b, o2b))
        sin = (sia, sib)
        sout = (soa, sob)

        def start_in(k):
            q = qoff + k * _CHUNK
            return pltpu.async_copy(
                mask_hbm.at[pl.ds(img * plane + q, _CHUNK)],
                masks[k % 2], sin[k % 2])

        def start_out(k):
            q = qoff + k * _CHUNK
            return [
                pltpu.async_copy(
                    o, out_hbm.at[pl.ds((img * 3 + c) * plane + q, _CHUNK)],
                    sout[k % 2])
                for c, o in enumerate(outs[k % 2])
            ]

        hin = {0: start_in(0)}

        pltpu.sync_copy(imap_hbm, imap_v.at[pl.ds(0, n_imap)])
        pltpu.sync_copy(mesh_hbm, mesh_v)
        for g in range(_TPAD // _L):
            mi = imap_v[pl.ds(g * _L, _L)]
            mi = jnp.clip(mi, 0, 215)
            for c, t in enumerate((t0, t1, t2)):
                cc = jnp.full((_L,), c, jnp.int32)
                t[pl.ds(g * _L, _L)] = plsc.load_gather(mesh_v, [mi, cc])
        hout = {}
        for k in range(n_chunks):
            p = k % 2
            if k + 1 < n_chunks:
                hin[k + 1] = start_in(k + 1)
            hin[k].wait()
            if k >= 2:
                for h in hout[k - 2]:
                    h.wait()
            mask_c, (o0, o1, o2) = masks[p], outs[p]

            @plsc.parallel_loop(0, _CHUNK // _L, unroll=8)
            def group(g):
                sl = pl.ds(g * _L, _L)
                idx = mask_c[sl]
                o0[sl] = plsc.load_gather(t0, [idx])
                o1[sl] = plsc.load_gather(t1, [idx])
                o2[sl] = plsc.load_gather(t2, [idx])

            hout[k] = start_out(k)
        for k in (n_chunks - 2, n_chunks - 1):
            for h in hout[k]:
                h.wait()

    return lookup


def kernel(mask, mesh_weight, index_map):
    n_img, n_rows, n_cols = mask.shape
    plane = n_rows * n_cols
    mask_phys = (mask.reshape(n_img, n_rows // 8, 8, n_cols // 128, 128)
                 .transpose(0, 1, 3, 2, 4).reshape(-1))
    out_flat = _make_lookup(n_img, plane, index_map.shape[0])(
        mask_phys, mesh_weight, index_map)
    out = (out_flat.reshape(n_img, 3, n_rows // 8, n_cols // 128, 8, 128)
           .transpose(0, 2, 4, 3, 5, 1).reshape(n_img, n_rows, n_cols, 3))
    return out


# speedup vs baseline: 1.0778x; 1.0778x over previous
import functools

import jax
import jax.numpy as jnp
from jax import lax
from jax.experimental import pallas as pl
from jax.experimental.pallas import tpu as pltpu
from jax.experimental.pallas import tpu_sc as plsc

_NC = 2
_NS = 16
_NW = _NC * _NS
_L = 16
_TPAD = 160
_CHUNK = 8192


def _make_lookup(n_img, plane, n_imap):
    per_tile = (n_img * plane) // _NW
    tiles_per_img = plane // per_tile
    n_chunks = per_tile // _CHUNK
    assert per_tile % _CHUNK == 0 and plane % per_tile == 0

    mesh = plsc.VectorSubcoreMesh(core_axis_name="c", subcore_axis_name="s")

    @functools.partial(
        pl.kernel,
        out_type=jax.ShapeDtypeStruct((n_img * 3 * plane,), jnp.float32),
        mesh=mesh,
        compiler_params=pltpu.CompilerParams(needs_layout_passes=False),
        scratch_types=[
            pltpu.VMEM((_TPAD,), jnp.int32),
            pltpu.VMEM((216 * 3,), jnp.float32),
            pltpu.VMEM((_TPAD,), jnp.float32),
            pltpu.VMEM((_TPAD,), jnp.float32),
            pltpu.VMEM((_TPAD,), jnp.float32),
            pltpu.VMEM((_CHUNK,), jnp.int32),
            pltpu.VMEM((_CHUNK,), jnp.int32),
            pltpu.VMEM((_CHUNK,), jnp.float32),
            pltpu.VMEM((_CHUNK,), jnp.float32),
            pltpu.VMEM((_CHUNK,), jnp.float32),
            pltpu.VMEM((_CHUNK,), jnp.float32),
            pltpu.VMEM((_CHUNK,), jnp.float32),
            pltpu.VMEM((_CHUNK,), jnp.float32),
            pltpu.SemaphoreType.DMA,
            pltpu.SemaphoreType.DMA,
            pltpu.SemaphoreType.DMA,
            pltpu.SemaphoreType.DMA,
        ],
    )
    def lookup(mask_hbm, mesh_hbm, imap_hbm, out_hbm,
               imap_v, mesh_v, t0, t1, t2, mca, mcb,
               o0a, o1a, o2a, o0b, o1b, o2b, sia, sib, soa, sob):
        wid = lax.axis_index("s") * _NC + lax.axis_index("c")
        img = wid // tiles_per_img
        qoff = (wid % tiles_per_img) * per_tile
        masks = (mca, mcb)
        outs = ((o0a, o1a, o2a), (o0b, o1b, o2b))
        sin = (sia, sib)
        sout = (soa, sob)

        def start_in(k):
            q = qoff + k * _CHUNK
            return pltpu.async_copy(
                mask_hbm.at[pl.ds(img * plane + q, _CHUNK)],
                masks[k % 2], sin[k % 2])

        def start_out(k):
            q = qoff + k * _CHUNK
            return [
                pltpu.async_copy(
                    o, out_hbm.at[pl.ds((img * 3 + c) * plane + q, _CHUNK)],
                    sout[k % 2])
                for c, o in enumerate(outs[k % 2])
            ]

        hin = {0: start_in(0)}

        pltpu.sync_copy(imap_hbm, imap_v.at[pl.ds(0, n_imap)])
        pltpu.sync_copy(mesh_hbm, mesh_v)
        for g in range(_TPAD // _L):
            mi = imap_v[pl.ds(g * _L, _L)]
            mi3 = jnp.clip(mi, 0, 215) * 3
            for c, t in enumerate((t0, t1, t2)):
                t[pl.ds(g * _L, _L)] = plsc.load_gather(mesh_v, [mi3 + c])
        hout = {}
        for k in range(n_chunks):
            p = k % 2
            if k + 1 < n_chunks:
                hin[k + 1] = start_in(k + 1)
            hin[k].wait()
            if k >= 2:
                for h in hout[k - 2]:
                    h.wait()
            mask_c, (o0, o1, o2) = masks[p], outs[p]

            @plsc.parallel_loop(0, _CHUNK // _L, unroll=8)
            def group(g):
                sl = pl.ds(g * _L, _L)
                idx = mask_c[sl]
                o0[sl] = plsc.load_gather(t0, [idx])
                o1[sl] = plsc.load_gather(t1, [idx])
                o2[sl] = plsc.load_gather(t2, [idx])

            hout[k] = start_out(k)
        for k in (n_chunks - 2, n_chunks - 1):
            for h in hout[k]:
                h.wait()

    return lookup


def kernel(mask, mesh_weight, index_map):
    n_img, n_rows, n_cols = mask.shape
    plane = n_rows * n_cols
    mask_phys = (mask.reshape(n_img, n_rows // 8, 8, n_cols // 128, 128)
                 .transpose(0, 1, 3, 2, 4).reshape(-1))
    out_flat = _make_lookup(n_img, plane, index_map.shape[0])(
        mask_phys, mesh_weight.reshape(-1), index_map)
    out = (out_flat.reshape(n_img, 3, n_rows // 8, n_cols // 128, 8, 128)
           .transpose(0, 2, 4, 3, 5, 1).reshape(n_img, n_rows, n_cols, 3))
    return out


# speedup vs baseline: 1.0856x; 1.0072x over previous
import functools

import jax
import jax.numpy as jnp
from jax import lax
from jax.experimental import pallas as pl
from jax.experimental.pallas import tpu as pltpu
from jax.experimental.pallas import tpu_sc as plsc

_NC = 2
_NS = 16
_NW = _NC * _NS
_L = 16
_TPAD = 160
_CHUNK = 8192


def _make_lookup(n_img, plane, n_imap):
    per_tile = (n_img * plane) // _NW
    tiles_per_img = plane // per_tile
    n_chunks = per_tile // _CHUNK
    assert per_tile % _CHUNK == 0 and plane % per_tile == 0

    mesh = plsc.VectorSubcoreMesh(core_axis_name="c", subcore_axis_name="s")

    @functools.partial(
        pl.kernel,
        out_type=jax.ShapeDtypeStruct((n_img * 3 * plane,), jnp.float32),
        mesh=mesh,
        compiler_params=pltpu.CompilerParams(needs_layout_passes=False),
        scratch_types=[
            pltpu.VMEM((_TPAD,), jnp.int32),
            pltpu.VMEM((216 * 3,), jnp.float32),
            pltpu.VMEM((_TPAD,), jnp.float32),
            pltpu.VMEM((_TPAD,), jnp.float32),
            pltpu.VMEM((_TPAD,), jnp.float32),
            pltpu.VMEM((_CHUNK,), jnp.int32),
            pltpu.VMEM((_CHUNK,), jnp.int32),
            pltpu.VMEM((_CHUNK,), jnp.float32),
            pltpu.VMEM((_CHUNK,), jnp.float32),
            pltpu.VMEM((_CHUNK,), jnp.float32),
            pltpu.VMEM((_CHUNK,), jnp.float32),
            pltpu.VMEM((_CHUNK,), jnp.float32),
            pltpu.VMEM((_CHUNK,), jnp.float32),
            pltpu.SemaphoreType.DMA,
            pltpu.SemaphoreType.DMA,
            pltpu.SemaphoreType.DMA,
            pltpu.SemaphoreType.DMA,
        ],
    )
    def lookup(mask_hbm, mesh_hbm, imap_hbm, out_hbm,
               imap_v, mesh_v, t0, t1, t2, mca, mcb,
               o0a, o1a, o2a, o0b, o1b, o2b, sia, sib, soa, sob):
        wid = lax.axis_index("s") * _NC + lax.axis_index("c")
        img = wid // tiles_per_img
        qoff = (wid % tiles_per_img) * per_tile
        masks = (mca, mcb)
        outs = ((o0a, o1a, o2a), (o0b, o1b, o2b))
        sin = (sia, sib)
        sout = (soa, sob)

        def start_in(k):
            q = qoff + k * _CHUNK
            return pltpu.async_copy(
                mask_hbm.at[pl.ds(img * plane + q, _CHUNK)],
                masks[k % 2], sin[k % 2])

        def start_out(k):
            q = qoff + k * _CHUNK
            return [
                pltpu.async_copy(
                    o, out_hbm.at[pl.ds((img * 3 + c) * plane + q, _CHUNK)],
                    sout[k % 2])
                for c, o in enumerate(outs[k % 2])
            ]

        hin = {0: start_in(0)}

        pltpu.sync_copy(imap_hbm, imap_v.at[pl.ds(0, n_imap)])
        pltpu.sync_copy(mesh_hbm, mesh_v)
        for g in range(_TPAD // _L):
            mi = imap_v[pl.ds(g * _L, _L)]
            mi3 = jnp.clip(mi, 0, 215) * 3
            for c, t in enumerate((t0, t1, t2)):
                t[pl.ds(g * _L, _L)] = plsc.load_gather(mesh_v, [mi3 + c])
        hout = {}
        for k in range(n_chunks):
            p = k % 2
            if k + 1 < n_chunks:
                hin[k + 1] = start_in(k + 1)
            hin[k].wait()
            if k >= 2:
                for h in hout[k - 2]:
                    h.wait()
            mask_c, (o0, o1, o2) = masks[p], outs[p]

            @plsc.parallel_loop(0, _CHUNK // _L, unroll=4)
            def group(g):
                sl = pl.ds(g * _L, _L)
                idx = mask_c[sl]
                o0[sl] = plsc.load_gather(t0, [idx])
                o1[sl] = plsc.load_gather(t1, [idx])
                o2[sl] = plsc.load_gather(t2, [idx])

            hout[k] = start_out(k)
        for k in (n_chunks - 2, n_chunks - 1):
            for h in hout[k]:
                h.wait()

    return lookup


def kernel(mask, mesh_weight, index_map):
    n_img, n_rows, n_cols = mask.shape
    plane = n_rows * n_cols
    mask_phys = (mask.reshape(n_img, n_rows // 8, 8, n_cols // 128, 128)
                 .transpose(0, 1, 3, 2, 4).reshape(-1))
    out_flat = _make_lookup(n_img, plane, index_map.shape[0])(
        mask_phys, mesh_weight.reshape(-1), index_map)
    out = (out_flat.reshape(n_img, 3, n_rows // 8, n_cols // 128, 8, 128)
           .transpose(0, 2, 4, 3, 5, 1).reshape(n_img, n_rows, n_cols, 3))
    return out
